# baseline (speedup 1.0000x reference)
import functools

import jax
import jax.numpy as jnp
from jax import lax
from jax.experimental import pallas as pl
from jax.experimental.pallas import tpu as pltpu
from jax.experimental.pallas import tpu_sc as plsc

D_FEAT = 128
D_POS = 3
D_PAD = 256
BN = 1000

NC = 2
NS = 16
NW = NC * NS
L = 16
NK = D_FEAT // L

CHUNK = 8000
GB = 128
MBUF = CHUNK + GB + L


def _mm_body(xp_ref, w_ref, b_ref, y_ref, z_ref):
    acc = jnp.dot(xp_ref[...], w_ref[...], preferred_element_type=jnp.float32)
    y_ref[...] = acc[:, :D_FEAT] + b_ref[...]
    z_ref[...] = acc[:, D_FEAT:]


def _make_mm(n_pad):
    grid = n_pad // BN
    return pl.pallas_call(
        _mm_body,
        grid=(grid,),
        in_specs=[
            pl.BlockSpec((BN, D_PAD), lambda i: (i, 0)),
            pl.BlockSpec((D_PAD, 2 * D_FEAT), lambda i: (0, 0)),
            pl.BlockSpec((1, D_FEAT), lambda i: (0, 0)),
        ],
        out_specs=[
            pl.BlockSpec((BN, D_FEAT), lambda i: (i, 0)),
            pl.BlockSpec((BN, D_FEAT), lambda i: (i, 0)),
        ],
        out_shape=[
            jax.ShapeDtypeStruct((n_pad, D_FEAT), jnp.float32),
            jax.ShapeDtypeStruct((n_pad, D_FEAT), jnp.float32),
        ],
    )


def _make_segmax(n_nodes, n_edges, npw):
    n_chunks = n_edges // CHUNK
    mesh = plsc.VectorSubcoreMesh(core_axis_name="c", subcore_axis_name="s")

    def body(y_hbm, z_hbm, src_hbm, dst_hbm, out_hbm,
             acc, dstb, srcb, msrc, mdst, rows, sem):
        wid = lax.axis_index("s") * NC + lax.axis_index("c")
        lo = wid * npw
        cnt = jnp.minimum(npw, n_nodes - lo)

        def init_row(r, carry):
            for k in range(NK):
                acc[r, pl.ds(k * L, L)] = jnp.full((L,), -jnp.inf, jnp.float32)
            return carry
        lax.fori_loop(0, npw, init_row, 0)

        def init_idx(i, carry):
            msrc[pl.ds(i * L, L)] = jnp.zeros((L,), jnp.int32)
            return carry
        lax.fori_loop(0, MBUF // L, init_idx, 0)

        def chunk_body(c, carry):
            pltpu.sync_copy(dst_hbm.at[pl.ds(c * CHUNK, CHUNK)], dstb)
            pltpu.sync_copy(src_hbm.at[pl.ds(c * CHUNK, CHUNK)], srcb)

            def scan_body(i, cur):
                dv = dstb[pl.ds(i * L, L)]
                sv = srcb[pl.ds(i * L, L)]
                m = (dv >= lo) & (dv < lo + npw)
                ones = jnp.where(m, 1, 0).astype(jnp.int32)
                incl = jnp.cumsum(ones)
                posv = cur + incl - 1
                plsc.store_scatter(msrc, [posv], sv, mask=m)
                plsc.store_scatter(mdst, [posv], dv - lo, mask=m)
                return cur + incl[L - 1]
            matched = lax.fori_loop(0, CHUNK // L, scan_body, 0,
                                    unroll=4)

            nb = (matched + GB - 1) // GB

            def batch_body(bi, carry):
                pltpu.async_copy(
                    y_hbm.at[msrc.at[pl.ds(bi * GB, GB)]], rows, sem).wait()
                nmax = jnp.minimum(GB, matched - bi * GB)

                def edge_body(e, carry2):
                    d = mdst[pl.ds(bi * GB + e, L)][0]
                    for k in range(NK):
                        sl = pl.ds(k * L, L)
                        acc[d, sl] = jnp.maximum(acc[d, sl], rows[e, sl])
                    return carry2
                lax.fori_loop(0, nmax, edge_body, 0)
                return carry
            lax.fori_loop(0, nb, batch_body, 0)
            return carry
        lax.fori_loop(0, n_chunks, chunk_body, 0)

        n_strips = cnt // 80

        def strip_body(s, carry):
            off = s * 80
            pltpu.sync_copy(z_hbm.at[pl.ds(lo + off, 80)],
                            rows.at[pl.ds(0, 80)])

            def row_body(r, carry2):
                for k in range(NK):
                    sl = pl.ds(k * L, L)
                    v = acc[off + r, sl] - rows[r, sl]
                    acc[off + r, sl] = jnp.maximum(v, 0.0)
                return carry2
            lax.fori_loop(0, 80, row_body, 0)
            pltpu.sync_copy(acc.at[pl.ds(off, 80)],
                            out_hbm.at[pl.ds(lo + off, 80)])
            return carry
        lax.fori_loop(0, n_strips, strip_body, 0)

    return pl.kernel(
        body,
        out_type=jax.ShapeDtypeStruct((n_nodes, D_FEAT), jnp.float32),
        mesh=mesh,
        compiler_params=pltpu.CompilerParams(needs_layout_passes=False),
        scratch_types=[
            pltpu.VMEM((npw, D_FEAT), jnp.float32),
            pltpu.VMEM((CHUNK,), jnp.int32),
            pltpu.VMEM((CHUNK,), jnp.int32),
            pltpu.VMEM((MBUF,), jnp.int32),
            pltpu.VMEM((MBUF,), jnp.int32),
            pltpu.VMEM((GB, D_FEAT), jnp.float32),
            pltpu.SemaphoreType.DMA,
        ],
    )


@jax.jit
def kernel(x, pos, edge_index, batch, W, b):
    n = x.shape[0]
    e = edge_index.shape[1]
    npw = -(-n // NW)
    npw = -(-npw // 80) * 80

    n_pad = -(-n // BN) * BN
    xp = jnp.concatenate(
        [x, pos, jnp.zeros((n, D_PAD - D_FEAT - D_POS), x.dtype)], axis=1)
    if n_pad != n:
        xp = jnp.pad(xp, ((0, n_pad - n), (0, 0)))

    zpad = jnp.zeros((D_PAD - D_FEAT - D_POS, D_FEAT), W.dtype)
    wy = jnp.concatenate([W, zpad], axis=0)
    wz = jnp.concatenate(
        [jnp.zeros((D_FEAT, D_FEAT), W.dtype), W[D_FEAT:], zpad], axis=0)
    wcat = jnp.concatenate([wy, wz], axis=1)

    y, z = _make_mm(n_pad)(xp, wcat, b.reshape(1, D_FEAT))
    y = y[:n]
    z = z[:n]

    src = edge_index[0]
    dst = edge_index[1]
    e_pad = -(-e // CHUNK) * CHUNK
    if e_pad != e:
        src = jnp.pad(src, (0, e_pad - e))
        dst = jnp.pad(dst, (0, e_pad - e), constant_values=npw * NW)

    out = _make_segmax(n, e_pad, npw)(y, z, src, dst)
    return (out, pos, batch)


# speedup vs baseline: 2.2676x; 2.2676x over previous
import functools

import jax
import jax.numpy as jnp
from jax import lax
from jax.experimental import pallas as pl
from jax.experimental.pallas import tpu as pltpu
from jax.experimental.pallas import tpu_sc as plsc

D_FEAT = 128
D_POS = 3
D_PAD = 256
BN = 1000

NC = 2
NS = 16
NW = NC * NS
L = 16
NK = D_FEAT // L

CHUNK = 8000
GB = 64
RB = 4
MBUF = CHUNK + GB + L


def _mm_body(xp_ref, w_ref, b_ref, y_ref, z_ref):
    acc = jnp.dot(xp_ref[...], w_ref[...], preferred_element_type=jnp.float32)
    y_ref[...] = acc[:, :D_FEAT] + b_ref[...]
    z_ref[...] = acc[:, D_FEAT:]


def _make_mm(n_pad):
    grid = n_pad // BN
    return pl.pallas_call(
        _mm_body,
        grid=(grid,),
        in_specs=[
            pl.BlockSpec((BN, D_PAD), lambda i: (i, 0)),
            pl.BlockSpec((D_PAD, 2 * D_FEAT), lambda i: (0, 0)),
            pl.BlockSpec((1, D_FEAT), lambda i: (0, 0)),
        ],
        out_specs=[
            pl.BlockSpec((BN, D_FEAT), lambda i: (i, 0)),
            pl.BlockSpec((BN, D_FEAT), lambda i: (i, 0)),
        ],
        out_shape=[
            jax.ShapeDtypeStruct((n_pad, D_FEAT), jnp.float32),
            jax.ShapeDtypeStruct((n_pad, D_FEAT), jnp.float32),
        ],
    )


def _make_segmax(n_nodes, n_edges, npw):
    n_chunks = n_edges // CHUNK
    mesh = plsc.VectorSubcoreMesh(core_axis_name="c", subcore_axis_name="s")

    def body(y_hbm, z_hbm, src_hbm, dst_hbm, out_hbm,
             acc, dstb, srcb, msrc, mdst, rows, sem):
        wid = lax.axis_index("s") * NC + lax.axis_index("c")
        lo = wid * npw
        cnt = jnp.minimum(npw, n_nodes - lo)

        def init_row(r, carry):
            for k in range(NK):
                acc[r, pl.ds(k * L, L)] = jnp.full((L,), -jnp.inf, jnp.float32)
            return carry
        lax.fori_loop(0, npw, init_row, 0)

        def init_idx(i, carry):
            msrc[pl.ds(i * L, L)] = jnp.zeros((L,), jnp.int32)
            return carry
        lax.fori_loop(0, MBUF // L, init_idx, 0)

        def chunk_body(c, carry):
            pltpu.sync_copy(dst_hbm.at[pl.ds(c * CHUNK, CHUNK)], dstb)
            pltpu.sync_copy(src_hbm.at[pl.ds(c * CHUNK, CHUNK)], srcb)

            def scan_body(i, cur):
                dv = dstb[pl.ds(i * L, L)]
                sv = srcb[pl.ds(i * L, L)]
                m = (dv >= lo) & (dv < lo + npw)
                ones = jnp.where(m, 1, 0).astype(jnp.int32)
                incl = jnp.cumsum(ones)
                posv = cur + incl - 1
                plsc.store_scatter(msrc, [posv], sv, mask=m)
                plsc.store_scatter(mdst, [posv], dv - lo, mask=m)
                return cur + incl[L - 1]
            matched = lax.fori_loop(0, CHUNK // L, scan_body, 0,
                                    unroll=4)

            nb = (matched + GB - 1) // GB

            def fire(bi):
                slot = lax.rem(bi, RB)
                pltpu.async_copy(
                    y_hbm.at[msrc.at[pl.ds(bi * GB, GB)]],
                    rows.at[pl.ds(slot * GB, GB)], sem)

            for j in range(RB):
                @pl.when(j < nb)
                def _():
                    fire(j)

            def batch_body(bi, carry):
                pltpu.make_async_copy(
                    y_hbm.at[pl.ds(0, GB)],
                    rows.at[pl.ds(0, GB)], sem).wait()
                nmax = jnp.minimum(GB, matched - bi * GB)
                base = lax.rem(bi, RB) * GB

                def edge_body(e, carry2):
                    d = mdst[pl.ds(bi * GB + e, L)][0]
                    for k in range(NK):
                        sl = pl.ds(k * L, L)
                        acc[d, sl] = jnp.maximum(acc[d, sl],
                                                 rows[base + e, sl])
                    return carry2
                lax.fori_loop(0, nmax, edge_body, 0)

                @pl.when(bi + RB < nb)
                def _():
                    fire(bi + RB)
                return carry
            lax.fori_loop(0, nb, batch_body, 0)
            return carry
        lax.fori_loop(0, n_chunks, chunk_body, 0)

        n_strips = cnt // 80

        def strip_body(s, carry):
            off = s * 80
            pltpu.sync_copy(z_hbm.at[pl.ds(lo + off, 80)],
                            rows.at[pl.ds(0, 80)])

            def row_body(r, carry2):
                for k in range(NK):
                    sl = pl.ds(k * L, L)
                    v = acc[off + r, sl] - rows[r, sl]
                    acc[off + r, sl] = jnp.maximum(v, 0.0)
                return carry2
            lax.fori_loop(0, 80, row_body, 0)
            pltpu.sync_copy(acc.at[pl.ds(off, 80)],
                            out_hbm.at[pl.ds(lo + off, 80)])
            return carry
        lax.fori_loop(0, n_strips, strip_body, 0)

    return pl.kernel(
        body,
        out_type=jax.ShapeDtypeStruct((n_nodes, D_FEAT), jnp.float32),
        mesh=mesh,
        compiler_params=pltpu.CompilerParams(needs_layout_passes=False),
        scratch_types=[
            pltpu.VMEM((npw, D_FEAT), jnp.float32),
            pltpu.VMEM((CHUNK,), jnp.int32),
            pltpu.VMEM((CHUNK,), jnp.int32),
            pltpu.VMEM((MBUF,), jnp.int32),
            pltpu.VMEM((MBUF,), jnp.int32),
            pltpu.VMEM((RB * GB, D_FEAT), jnp.float32),
            pltpu.SemaphoreType.DMA,
        ],
    )


@jax.jit
def kernel(x, pos, edge_index, batch, W, b):
    n = x.shape[0]
    e = edge_index.shape[1]
    npw = -(-n // NW)
    npw = -(-npw // 80) * 80

    n_pad = -(-n // BN) * BN
    xp = jnp.concatenate(
        [x, pos, jnp.zeros((n, D_PAD - D_FEAT - D_POS), x.dtype)], axis=1)
    if n_pad != n:
        xp = jnp.pad(xp, ((0, n_pad - n), (0, 0)))

    zpad = jnp.zeros((D_PAD - D_FEAT - D_POS, D_FEAT), W.dtype)
    wy = jnp.concatenate([W, zpad], axis=0)
    wz = jnp.concatenate(
        [jnp.zeros((D_FEAT, D_FEAT), W.dtype), W[D_FEAT:], zpad], axis=0)
    wcat = jnp.concatenate([wy, wz], axis=1)

    y, z = _make_mm(n_pad)(xp, wcat, b.reshape(1, D_FEAT))
    y = y[:n]
    z = z[:n]

    src = edge_index[0]
    dst = edge_index[1]
    e_pad = -(-e // CHUNK) * CHUNK
    if e_pad != e:
        src = jnp.pad(src, (0, e_pad - e))
        dst = jnp.pad(dst, (0, e_pad - e), constant_values=npw * NW)

    out = _make_segmax(n, e_pad, npw)(y, z, src, dst)
    return (out, pos, batch)
